# baseline (speedup 1.0000x reference)
import functools

import jax
import jax.numpy as jnp
from jax import lax
from jax.experimental import pallas as pl
from jax.experimental.pallas import tpu as pltpu
from jax.experimental.pallas import tpu_sc as plsc

VOCAB = 1000000
D = 32
BATCH = 4096
SEQ = 200
N = BATCH * SEQ

NUM_CORES = 2
NUM_SUBCORES = 16
NW = NUM_CORES * NUM_SUBCORES
BROWS_PER_W = BATCH // NW
CHUNK = SEQ
NCHUNKS = BROWS_PER_W
NPAIR = NCHUNKS // 2
L = 16
NG = SEQ // L
TAIL = SEQ - L
GROWS = (NG + 1) * L

_mesh = plsc.VectorSubcoreMesh(core_axis_name="c", subcore_axis_name="s")


@functools.partial(
    pl.kernel,
    mesh=_mesh,
    compiler_params=pltpu.CompilerParams(use_tc_tiling_on_sc=False),
    out_type=jax.ShapeDtypeStruct((BATCH, SEQ, 2 * D), jnp.float32),
    scratch_types=[
        pltpu.VMEM((BROWS_PER_W, SEQ), jnp.int32),
        pltpu.VMEM((BROWS_PER_W, SEQ), jnp.int32),
        pltpu.VMEM((SEQ, D), jnp.float32),
        pltpu.VMEM((SEQ, D), jnp.float32),
        pltpu.VMEM((SEQ, D), jnp.float32),
        pltpu.VMEM((SEQ, D), jnp.float32),
        pltpu.SemaphoreType.DMA,
        pltpu.SemaphoreType.DMA,
        pltpu.SemaphoreType.DMA,
        pltpu.SemaphoreType.DMA,
    ],
)
def _embed_cat(wid_hbm, cid_hbm, ww_hbm, wc_hbm, out3_hbm,
               widx_v, cidx_v, wrow0, wrow1, crow0, crow1,
               gsem0, gsem1, wsem0, wsem1):
    w = lax.axis_index("s") * NUM_CORES + lax.axis_index("c")
    rbase = w * BROWS_PER_W
    wrow = (wrow0, wrow1)
    crow = (crow0, crow1)
    gsem = (gsem0, gsem1)
    wsem = (wsem0, wsem1)

    pltpu.sync_copy(wid_hbm.at[pl.ds(rbase, BROWS_PER_W)], widx_v)
    pltpu.sync_copy(cid_hbm.at[pl.ds(rbase, BROWS_PER_W)], cidx_v)

    def enqueue_gathers(i, b):
        for j in range(NG + 1):
            sl = pl.ds(min(j * L, TAIL), L)
            pltpu.async_copy(ww_hbm.at[widx_v[i, sl]], wrow[b].at[sl], gsem[b])
            pltpu.async_copy(wc_hbm.at[cidx_v[i, sl]], crow[b].at[sl], gsem[b])

    def drain_gathers(b):
        for r in (wrow[b], crow[b]):
            pltpu.make_async_copy(ww_hbm.at[pl.ds(0, GROWS - L)],
                                  r.at[pl.ds(0, GROWS - L)], gsem[b]).wait()
            pltpu.make_async_copy(ww_hbm.at[pl.ds(0, L)],
                                  r.at[pl.ds(0, L)], gsem[b]).wait()

    def issue_writes(i, b):
        r = rbase + i
        pltpu.async_copy(wrow[b], out3_hbm.at[r, :, pl.ds(0, D)], wsem[b])
        pltpu.async_copy(crow[b], out3_hbm.at[r, :, pl.ds(D, D)], wsem[b])

    def wait_writes(b):
        pltpu.make_async_copy(
            wrow[b], out3_hbm.at[rbase, :, pl.ds(0, D)], wsem[b]).wait()
        pltpu.make_async_copy(
            crow[b], out3_hbm.at[rbase, :, pl.ds(D, D)], wsem[b]).wait()

    enqueue_gathers(0, 0)
    enqueue_gathers(1, 1)
    drain_gathers(0)
    issue_writes(0, 0)
    drain_gathers(1)
    issue_writes(1, 1)

    def pair(k, _):
        i0 = 2 * k + 2
        wait_writes(0)
        enqueue_gathers(i0, 0)
        wait_writes(1)
        enqueue_gathers(i0 + 1, 1)
        drain_gathers(0)
        issue_writes(i0, 0)
        drain_gathers(1)
        issue_writes(i0 + 1, 1)
        return ()

    lax.fori_loop(0, NPAIR - 1, pair, ())
    wait_writes(0)
    wait_writes(1)


def kernel(word_ids, char_ids, W_words, W_chars):
    wid = word_ids.astype(jnp.int32)
    cid = char_ids.astype(jnp.int32)
    return _embed_cat(wid, cid, W_words, W_chars)


# speedup vs baseline: 1.0002x; 1.0002x over previous
import functools

import jax
import jax.numpy as jnp
from jax import lax
from jax.experimental import pallas as pl
from jax.experimental.pallas import tpu as pltpu
from jax.experimental.pallas import tpu_sc as plsc

VOCAB = 1000000
D = 32
BATCH = 4096
SEQ = 200
N = BATCH * SEQ

NUM_CORES = 2
NUM_SUBCORES = 16
NW = NUM_CORES * NUM_SUBCORES
BROWS_PER_W = BATCH // NW
CHUNK = SEQ
NCHUNKS = BROWS_PER_W
NPAIR = NCHUNKS // 2
L = 16
NG = SEQ // L
TAIL = SEQ - L
GROWS = (NG + 1) * L

_mesh = plsc.VectorSubcoreMesh(core_axis_name="c", subcore_axis_name="s")


@functools.partial(
    pl.kernel,
    mesh=_mesh,
    compiler_params=pltpu.CompilerParams(use_tc_tiling_on_sc=False),
    out_type=jax.ShapeDtypeStruct((BATCH, SEQ, 2 * D), jnp.float32),
    scratch_types=[
        pltpu.VMEM((BROWS_PER_W, 256), jnp.int32),
        pltpu.VMEM((BROWS_PER_W, 256), jnp.int32),
        pltpu.VMEM((SEQ, D), jnp.float32),
        pltpu.VMEM((SEQ, D), jnp.float32),
        pltpu.VMEM((SEQ, D), jnp.float32),
        pltpu.VMEM((SEQ, D), jnp.float32),
        pltpu.SemaphoreType.DMA,
        pltpu.SemaphoreType.DMA,
        pltpu.SemaphoreType.DMA,
        pltpu.SemaphoreType.DMA,
    ],
)
def _embed_cat(wid_hbm, cid_hbm, ww_hbm, wc_hbm, out3_hbm,
               widx_v, cidx_v, wrow0, wrow1, crow0, crow1,
               gsem0, gsem1, wsem0, wsem1):
    w = lax.axis_index("s") * NUM_CORES + lax.axis_index("c")
    rbase = w * BROWS_PER_W
    wrow = (wrow0, wrow1)
    crow = (crow0, crow1)
    gsem = (gsem0, gsem1)
    wsem = (wsem0, wsem1)

    pltpu.sync_copy(wid_hbm.at[pl.ds(rbase, BROWS_PER_W)], widx_v)
    pltpu.sync_copy(cid_hbm.at[pl.ds(rbase, BROWS_PER_W)], cidx_v)

    def enqueue_gathers(i, b):
        for j in range(NG + 1):
            sl = pl.ds(min(j * L, TAIL), L)
            pltpu.async_copy(ww_hbm.at[widx_v[i, sl]], wrow[b].at[sl], gsem[b])
            pltpu.async_copy(wc_hbm.at[cidx_v[i, sl]], crow[b].at[sl], gsem[b])

    def drain_gathers(b):
        for r in (wrow[b], crow[b]):
            pltpu.make_async_copy(ww_hbm.at[pl.ds(0, GROWS - L)],
                                  r.at[pl.ds(0, GROWS - L)], gsem[b]).wait()
            pltpu.make_async_copy(ww_hbm.at[pl.ds(0, L)],
                                  r.at[pl.ds(0, L)], gsem[b]).wait()

    def issue_writes(i, b):
        r = rbase + i
        pltpu.async_copy(wrow[b], out3_hbm.at[r, :, pl.ds(0, D)], wsem[b])
        pltpu.async_copy(crow[b], out3_hbm.at[r, :, pl.ds(D, D)], wsem[b])

    def wait_writes(b):
        pltpu.make_async_copy(
            wrow[b], out3_hbm.at[rbase, :, pl.ds(0, D)], wsem[b]).wait()
        pltpu.make_async_copy(
            crow[b], out3_hbm.at[rbase, :, pl.ds(D, D)], wsem[b]).wait()

    enqueue_gathers(0, 0)
    enqueue_gathers(1, 1)
    drain_gathers(0)
    issue_writes(0, 0)
    drain_gathers(1)
    issue_writes(1, 1)

    def pair(k, _):
        i0 = 2 * k + 2
        wait_writes(0)
        enqueue_gathers(i0, 0)
        wait_writes(1)
        enqueue_gathers(i0 + 1, 1)
        drain_gathers(0)
        issue_writes(i0, 0)
        drain_gathers(1)
        issue_writes(i0 + 1, 1)
        return ()

    lax.fori_loop(0, NPAIR - 1, pair, ())
    wait_writes(0)
    wait_writes(1)


def kernel(word_ids, char_ids, W_words, W_chars):
    pad = ((0, 0), (0, 56))
    wid = jnp.pad(word_ids.astype(jnp.int32), pad)
    cid = jnp.pad(char_ids.astype(jnp.int32), pad)
    return _embed_cat(wid, cid, W_words, W_chars)


# speedup vs baseline: 1.0126x; 1.0124x over previous
import functools

import jax
import jax.numpy as jnp
from jax import lax
from jax.experimental import pallas as pl
from jax.experimental.pallas import tpu as pltpu
from jax.experimental.pallas import tpu_sc as plsc

VOCAB = 1000000
D = 32
BATCH = 4096
SEQ = 200
N = BATCH * SEQ

NUM_CORES = 2
NUM_SUBCORES = 16
NW = NUM_CORES * NUM_SUBCORES
BROWS_PER_W = BATCH // NW
CHUNK = 2 * SEQ
NCHUNKS = BROWS_PER_W // 2
NPAIR = NCHUNKS // 2
L = 16
NG = CHUNK // L

_mesh = plsc.VectorSubcoreMesh(core_axis_name="c", subcore_axis_name="s")


@functools.partial(
    pl.kernel,
    mesh=_mesh,
    compiler_params=pltpu.CompilerParams(use_tc_tiling_on_sc=False),
    out_type=jax.ShapeDtypeStruct((BATCH, SEQ, 2 * D), jnp.float32),
    scratch_types=[
        pltpu.VMEM((NCHUNKS, CHUNK), jnp.int32),
        pltpu.VMEM((NCHUNKS, CHUNK), jnp.int32),
        pltpu.VMEM((CHUNK, D), jnp.float32),
        pltpu.VMEM((CHUNK, D), jnp.float32),
        pltpu.VMEM((CHUNK, D), jnp.float32),
        pltpu.VMEM((CHUNK, D), jnp.float32),
        pltpu.SemaphoreType.DMA,
        pltpu.SemaphoreType.DMA,
        pltpu.SemaphoreType.DMA,
        pltpu.SemaphoreType.DMA,
    ],
)
def _embed_cat(wid_hbm, cid_hbm, ww_hbm, wc_hbm, out3_hbm,
               widx2_v, cidx2_v, wrow0, wrow1, crow0, crow1,
               gsem0, gsem1, wsem0, wsem1):
    w = lax.axis_index("s") * NUM_CORES + lax.axis_index("c")
    rbase = w * BROWS_PER_W
    wrow = (wrow0, wrow1)
    crow = (crow0, crow1)
    gsem = (gsem0, gsem1)
    wsem = (wsem0, wsem1)

    pltpu.sync_copy(wid_hbm.at[pl.ds(w * NCHUNKS, NCHUNKS)], widx2_v)
    pltpu.sync_copy(cid_hbm.at[pl.ds(w * NCHUNKS, NCHUNKS)], cidx2_v)

    def enqueue_gathers(i, b):
        for j in range(NG):
            sl = pl.ds(j * L, L)
            pltpu.async_copy(ww_hbm.at[widx2_v[i, sl]], wrow[b].at[sl], gsem[b])
            pltpu.async_copy(wc_hbm.at[cidx2_v[i, sl]], crow[b].at[sl], gsem[b])

    def drain_gathers(b):
        pltpu.make_async_copy(ww_hbm.at[pl.ds(0, CHUNK)], wrow[b], gsem[b]).wait()
        pltpu.make_async_copy(wc_hbm.at[pl.ds(0, CHUNK)], crow[b], gsem[b]).wait()

    def issue_writes(i, b):
        r = rbase + 2 * i
        h0, h1 = pl.ds(0, SEQ), pl.ds(SEQ, SEQ)
        pltpu.async_copy(wrow[b].at[h0], out3_hbm.at[r, :, pl.ds(0, D)], wsem[b])
        pltpu.async_copy(wrow[b].at[h1], out3_hbm.at[r + 1, :, pl.ds(0, D)], wsem[b])
        pltpu.async_copy(crow[b].at[h0], out3_hbm.at[r, :, pl.ds(D, D)], wsem[b])
        pltpu.async_copy(crow[b].at[h1], out3_hbm.at[r + 1, :, pl.ds(D, D)], wsem[b])

    def wait_writes(b):
        h0 = pl.ds(0, SEQ)
        for _ in range(2):
            pltpu.make_async_copy(
                wrow[b].at[h0], out3_hbm.at[rbase, :, pl.ds(0, D)], wsem[b]).wait()
            pltpu.make_async_copy(
                crow[b].at[h0], out3_hbm.at[rbase, :, pl.ds(D, D)], wsem[b]).wait()

    enqueue_gathers(0, 0)
    enqueue_gathers(1, 1)
    drain_gathers(0)
    issue_writes(0, 0)
    drain_gathers(1)
    issue_writes(1, 1)

    def pair(k, _):
        i0 = 2 * k + 2
        wait_writes(0)
        enqueue_gathers(i0, 0)
        wait_writes(1)
        enqueue_gathers(i0 + 1, 1)
        drain_gathers(0)
        issue_writes(i0, 0)
        drain_gathers(1)
        issue_writes(i0 + 1, 1)
        return ()

    lax.fori_loop(0, NPAIR - 1, pair, ())
    wait_writes(0)
    wait_writes(1)


def kernel(word_ids, char_ids, W_words, W_chars):
    wid = word_ids.astype(jnp.int32).reshape(N // CHUNK, CHUNK)
    cid = char_ids.astype(jnp.int32).reshape(N // CHUNK, CHUNK)
    return _embed_cat(wid, cid, W_words, W_chars)
